# baseline (speedup 1.0000x reference)
import functools

import jax
import jax.numpy as jnp
from jax import lax
from jax.experimental import pallas as pl
from jax.experimental.pallas import tpu as pltpu
from jax.experimental.pallas import tpu_sc as plsc

_N = 10000
_E = 320000
_D = 128
_NC = 2
_NS = 16
_NW = _NC * _NS
_K = 128
_EPT = 10240
_EPAD = _NW * _EPT
_NCHUNK = _EPT // _K
_NP = 10240
_RPT = _NP // _NS


def _sc_segment_sum(h, src, dst, zrows):
  mesh = plsc.VectorSubcoreMesh(core_axis_name="c", subcore_axis_name="s",
                                num_cores=_NC, num_subcores=_NS)

  @functools.partial(
      pl.kernel,
      out_type=(jax.ShapeDtypeStruct((_NC, _NP, _D), jnp.float32),
                jax.ShapeDtypeStruct((_NW, _NP), jnp.float32)),
      mesh=mesh,
      scratch_types=[
          pltpu.VMEM_SHARED((_NP, _D), jnp.float32),
          pltpu.VMEM((_K,), jnp.int32),
          pltpu.VMEM((_K,), jnp.int32),
          pltpu.VMEM((_K,), jnp.int32),
          pltpu.VMEM((_K,), jnp.int32),
          pltpu.VMEM((_K, _D), jnp.float32),
          pltpu.VMEM((_K, _D), jnp.float32),
          pltpu.VMEM((_NP,), jnp.float32),
          pltpu.SemaphoreType.DMA,
          pltpu.SemaphoreType.DMA,
          pltpu.SemaphoreType.DMA,
          pltpu.SemaphoreType.DMA,
          pltpu.SemaphoreType.DMA,
          pltpu.SemaphoreType.DMA,
      ],
      compiler_params=pltpu.CompilerParams(needs_layout_passes=False),
  )
  def seg_sum(h_hbm, src_hbm, dst_hbm, zrows_hbm,
              agg_out, cnt_out, agg_sp,
              idx_s_a, idx_s_b, idx_d_a, idx_d_b, rows_a, rows_b, cnt_loc,
              sem_sa, sem_sb, sem_da, sem_db, sem_ra, sem_rb):
    c = lax.axis_index("c")
    s = lax.axis_index("s")
    wid = c * _NS + s
    r0 = s * _RPT
    base = wid * _EPT
    bufs = ((idx_s_a, idx_d_a, rows_a, sem_sa, sem_da, sem_ra),
            (idx_s_b, idx_d_b, rows_b, sem_sb, sem_db, sem_rb))

    def fetch_idx(g, bi):
      idx_s_buf, idx_d_buf, _, sem_s, sem_d, _ = bufs[bi]
      off = base + g * _K
      pltpu.async_copy(src_hbm.at[pl.ds(off, _K)], idx_s_buf, sem_s)
      pltpu.async_copy(dst_hbm.at[pl.ds(off, _K)], idx_d_buf, sem_d)

    fetch_idx(0, 0)
    fetch_idx(1, 1)

    pltpu.sync_copy(zrows_hbm, rows_a)
    for j in range(_RPT // _K):
      pltpu.sync_copy(rows_a, agg_sp.at[pl.ds(r0 + j * _K, _K)])
    zeros16 = jnp.zeros((16,), jnp.float32)

    def zi(i, carry):
      cnt_loc[pl.ds(i * 16, 16)] = zeros16
      return carry

    lax.fori_loop(0, _NP // 16, zi, 0)
    plsc.subcore_barrier()

    ones16 = jnp.ones((16,), jnp.float32)

    def drain_idx_s(bi):
      pltpu.make_async_copy(src_hbm.at[pl.ds(0, _K)], bufs[bi][0],
                            bufs[bi][3]).wait()

    def drain_idx_d(bi):
      pltpu.make_async_copy(dst_hbm.at[pl.ds(0, _K)], bufs[bi][1],
                            bufs[bi][4]).wait()

    def drain_rows(bi):
      pltpu.make_async_copy(h_hbm.at[pl.ds(0, _K)], bufs[bi][2],
                            bufs[bi][5]).wait()

    def start_gather(bi):
      idx_s_buf, _, rows_buf, _, _, sem_r = bufs[bi]
      pltpu.async_copy(h_hbm.at[idx_s_buf], rows_buf, sem_r)

    drain_idx_s(0)
    start_gather(0)

    def body(u, carry):
      for bi in range(2):
        g = 2 * u + bi
        nb = bi ^ 1
        ng = g + 1

        @pl.when(ng < _NCHUNK)
        def _():
          drain_idx_s(nb)
          start_gather(nb)

        drain_rows(bi)
        pg = g + 2

        @pl.when(pg < _NCHUNK)
        def _():
          idx_s_buf, _, _, sem_s, _, _ = bufs[bi]
          pltpu.async_copy(src_hbm.at[pl.ds(base + pg * _K, _K)],
                           idx_s_buf, sem_s)

        drain_idx_d(bi)
        pltpu.sync_copy(bufs[bi][2], agg_sp.at[bufs[bi][1]], add=True)
        for j in range(_K // 16):
          plsc.addupdate_scatter(cnt_loc, [bufs[bi][1][pl.ds(j * 16, 16)]],
                                 ones16)

        @pl.when(pg < _NCHUNK)
        def _():
          _, idx_d_buf, _, _, sem_d, _ = bufs[bi]
          pltpu.async_copy(dst_hbm.at[pl.ds(base + pg * _K, _K)],
                           idx_d_buf, sem_d)

      return carry

    lax.fori_loop(0, _NCHUNK // 2, body, 0)
    plsc.subcore_barrier()

    for j in range(_RPT // _K):
      pltpu.sync_copy(agg_sp.at[pl.ds(r0 + j * _K, _K)], rows_a)
      pltpu.sync_copy(rows_a, agg_out.at[c, pl.ds(r0 + j * _K, _K)])
    pltpu.sync_copy(cnt_loc, cnt_out.at[wid])

  return seg_sum(h, src, dst, zrows)


def _tc_self(h, Wr):
  def body(h_ref, wr_ref, out_ref):
    out_ref[...] = lax.dot_general(h_ref[...], wr_ref[...],
                                   (((1,), (1,)), ((), ())),
                                   preferred_element_type=jnp.float32)

  return pl.pallas_call(
      body,
      out_shape=jax.ShapeDtypeStruct((_N, _D), jnp.float32),
  )(h, Wr)


def _tc_merge(p, cntp, hr, Wl, bl, gw, gb, gms):
  def body(p_ref, cnt_ref, hr_ref, wl_ref, bl_ref, gw_ref, gb_ref,
           gms_ref, out_ref):
    agg = p_ref[0, :_N] + p_ref[1, :_N]
    cnt_row = jnp.sum(cnt_ref[...], axis=0, keepdims=True)
    cnt = jnp.transpose(cnt_row[:, :_N])
    agg = agg / jnp.maximum(cnt, 1.0)
    z = (lax.dot_general(agg, wl_ref[...], (((1,), (1,)), ((), ())),
                         preferred_element_type=jnp.float32)
         + bl_ref[...] + hr_ref[...])
    z = jnp.maximum(z, 0.0)
    mean = jnp.mean(z, axis=0, keepdims=True)
    out = z - mean * gms_ref[...]
    var = jnp.mean(out * out, axis=0, keepdims=True)
    out = out * lax.rsqrt(var + 1e-5)
    out_ref[...] = out * gw_ref[...] + gb_ref[...]

  return pl.pallas_call(
      body,
      out_shape=jax.ShapeDtypeStruct((_N, _D), jnp.float32),
  )(p, cntp, hr, Wl, bl, gw, gb, gms)


def kernel(x, edge_index, W1l, b1l, W1r, W2l, b2l, W2r,
           gn_weight, gn_bias, gn_mean_scale):
  src = edge_index[0].astype(jnp.int32)
  dst = edge_index[1].astype(jnp.int32)
  npad = _EPAD - _E
  src = jnp.concatenate([src, jnp.zeros((npad,), jnp.int32)])
  dst = jnp.concatenate([dst, jnp.full((npad,), _NP - 1, jnp.int32)])
  zrows = jnp.zeros((_K, _D), jnp.float32)
  gw = gn_weight.reshape(1, _D)
  gb = gn_bias.reshape(1, _D)
  gms = gn_mean_scale.reshape(1, _D)
  h = x
  for Wl, bl, Wr in ((W1l, b1l, W1r), (W2l, b2l, W2r)):
    hr = _tc_self(h, Wr)
    p, cntp = _sc_segment_sum(h, src, dst, zrows)
    h = _tc_merge(p, cntp, hr, Wl, bl.reshape(1, _D), gw, gb, gms)
  return h


# speedup vs baseline: 3.3814x; 3.3814x over previous
import functools

import jax
import jax.numpy as jnp
from jax import lax
from jax.experimental import pallas as pl
from jax.experimental.pallas import tpu as pltpu
from jax.experimental.pallas import tpu_sc as plsc

_N = 10000
_E = 320000
_D = 128
_NC = 2
_NS = 16
_NW = _NC * _NS
_EPT = _E // _NW
_K = 80
_NCHUNK = _EPT // _K
_NP = 10240
_RPT = _NP // _NS


def _sc_segment_sum(h, src, dst, zrows):
  mesh = plsc.VectorSubcoreMesh(core_axis_name="c", subcore_axis_name="s",
                                num_cores=_NC, num_subcores=_NS)

  @functools.partial(
      pl.kernel,
      out_type=(jax.ShapeDtypeStruct((_NC, _NP, _D), jnp.float32),
                jax.ShapeDtypeStruct((_NW, _NP), jnp.float32)),
      mesh=mesh,
      scratch_types=[
          pltpu.VMEM_SHARED((_NP, _D), jnp.float32),
          pltpu.VMEM((_EPT,), jnp.int32),
          pltpu.VMEM((_K,), jnp.int32),
          pltpu.VMEM((_K,), jnp.int32),
          pltpu.VMEM((_K, _D), jnp.float32),
          pltpu.VMEM((_K, _D), jnp.float32),
          pltpu.VMEM((_NP,), jnp.float32),
          pltpu.SemaphoreType.DMA,
          pltpu.SemaphoreType.DMA,
          pltpu.SemaphoreType.DMA,
          pltpu.SemaphoreType.DMA,
      ],
      compiler_params=pltpu.CompilerParams(needs_layout_passes=False),
  )
  def seg_sum(h_hbm, src_hbm, dst_hbm, zrows_hbm,
              agg_out, cnt_out, agg_sp, idx_all_s,
              idx_d_a, idx_d_b, rows_a, rows_b, cnt_loc,
              sem_ra, sem_rb, sem_ia, sem_ib):
    c = lax.axis_index("c")
    s = lax.axis_index("s")
    wid = c * _NS + s
    r0 = s * _RPT
    base = wid * _EPT
    pltpu.sync_copy(src_hbm.at[pl.ds(base, _EPT)], idx_all_s)
    pltpu.sync_copy(zrows_hbm, rows_a)
    for j in range(_RPT // _K):
      pltpu.sync_copy(rows_a, agg_sp.at[pl.ds(r0 + j * _K, _K)])
    zeros16 = jnp.zeros((16,), jnp.float32)

    def zi(i, carry):
      cnt_loc[pl.ds(i * 16, 16)] = zeros16
      return carry

    lax.fori_loop(0, _NP // 16, zi, 0)
    plsc.subcore_barrier()

    ones16 = jnp.ones((16,), jnp.float32)

    def fetch(g, idx_d_buf, rows_buf, sem_r, sem_i):
      pltpu.async_copy(dst_hbm.at[pl.ds(base + g * _K, _K)], idx_d_buf,
                       sem_i)
      pltpu.async_copy(h_hbm.at[idx_all_s.at[pl.ds(g * _K, _K)]],
                       rows_buf, sem_r)

    def wait_chunk(idx_d_buf, rows_buf, sem_r, sem_i):
      pltpu.make_async_copy(dst_hbm.at[pl.ds(0, _K)], idx_d_buf,
                            sem_i).wait()
      pltpu.make_async_copy(h_hbm.at[pl.ds(0, _K)], rows_buf, sem_r).wait()

    def hist(idx_d_buf):
      for j in range(_K // 16):
        plsc.addupdate_scatter(cnt_loc, [idx_d_buf[pl.ds(j * 16, 16)]],
                               ones16)

    fetch(0, idx_d_a, rows_a, sem_ra, sem_ia)
    fetch(1, idx_d_b, rows_b, sem_rb, sem_ib)

    def body(t, carry):
      for bi, (idx_d_buf, rows_buf, sem_r, sem_i) in enumerate(
          ((idx_d_a, rows_a, sem_ra, sem_ia),
           (idx_d_b, rows_b, sem_rb, sem_ib))):
        g = 2 * t + bi
        wait_chunk(idx_d_buf, rows_buf, sem_r, sem_i)
        pltpu.sync_copy(rows_buf, agg_sp.at[idx_d_buf], add=True)
        hist(idx_d_buf)
        pg = g + 2

        @pl.when(pg < _NCHUNK)
        def _():
          fetch(pg, idx_d_buf, rows_buf, sem_r, sem_i)

      return carry

    lax.fori_loop(0, _NCHUNK // 2, body, 0)
    wait_chunk(idx_d_a, rows_a, sem_ra, sem_ia)
    pltpu.sync_copy(rows_a, agg_sp.at[idx_d_a], add=True)
    hist(idx_d_a)
    plsc.subcore_barrier()

    for j in range(_RPT // _K):
      pltpu.sync_copy(agg_sp.at[pl.ds(r0 + j * _K, _K)], rows_a)
      pltpu.sync_copy(rows_a, agg_out.at[c, pl.ds(r0 + j * _K, _K)])
    pltpu.sync_copy(cnt_loc, cnt_out.at[wid])

  return seg_sum(h, src, dst, zrows)


def _tc_self(h, Wr):
  def body(h_ref, wr_ref, out_ref):
    out_ref[...] = lax.dot_general(h_ref[...], wr_ref[...],
                                   (((1,), (1,)), ((), ())),
                                   preferred_element_type=jnp.float32)

  return pl.pallas_call(
      body,
      out_shape=jax.ShapeDtypeStruct((_N, _D), jnp.float32),
  )(h, Wr)


def _tc_merge(p, cntp, hr, Wl, bl, gw, gb, gms):
  def body(p_ref, cnt_ref, hr_ref, wl_ref, bl_ref, gw_ref, gb_ref,
           gms_ref, out_ref):
    agg = p_ref[0, :_N] + p_ref[1, :_N]
    cnt_row = jnp.sum(cnt_ref[...], axis=0, keepdims=True)
    cnt = jnp.transpose(cnt_row[:, :_N])
    agg = agg / jnp.maximum(cnt, 1.0)
    z = (lax.dot_general(agg, wl_ref[...], (((1,), (1,)), ((), ())),
                         preferred_element_type=jnp.float32)
         + bl_ref[...] + hr_ref[...])
    z = jnp.maximum(z, 0.0)
    mean = jnp.mean(z, axis=0, keepdims=True)
    out = z - mean * gms_ref[...]
    var = jnp.mean(out * out, axis=0, keepdims=True)
    out = out * lax.rsqrt(var + 1e-5)
    out_ref[...] = out * gw_ref[...] + gb_ref[...]

  return pl.pallas_call(
      body,
      out_shape=jax.ShapeDtypeStruct((_N, _D), jnp.float32),
  )(p, cntp, hr, Wl, bl, gw, gb, gms)


def kernel(x, edge_index, W1l, b1l, W1r, W2l, b2l, W2r,
           gn_weight, gn_bias, gn_mean_scale):
  src = edge_index[0].astype(jnp.int32)
  dst = edge_index[1].astype(jnp.int32)
  zrows = jnp.zeros((_K, _D), jnp.float32)
  gw = gn_weight.reshape(1, _D)
  gb = gn_bias.reshape(1, _D)
  gms = gn_mean_scale.reshape(1, _D)
  h = x
  for Wl, bl, Wr in ((W1l, b1l, W1r), (W2l, b2l, W2r)):
    hr = _tc_self(h, Wr)
    p, cntp = _sc_segment_sum(h, src, dst, zrows)
    h = _tc_merge(p, cntp, hr, Wl, bl.reshape(1, _D), gw, gb, gms)
  return h


# speedup vs baseline: 3.3936x; 1.0036x over previous
import functools

import jax
import jax.numpy as jnp
from jax import lax
from jax.experimental import pallas as pl
from jax.experimental.pallas import tpu as pltpu
from jax.experimental.pallas import tpu_sc as plsc

_N = 10000
_E = 320000
_D = 128
_NC = 2
_NS = 16
_NW = _NC * _NS
_EPT = _E // _NW
_K = 80
_NCHUNK = _EPT // _K
_NP = 10240
_RPT = _NP // _NS


def _sc_segment_sum(h, src, dst, zrows):
  mesh = plsc.VectorSubcoreMesh(core_axis_name="c", subcore_axis_name="s",
                                num_cores=_NC, num_subcores=_NS)

  @functools.partial(
      pl.kernel,
      out_type=(jax.ShapeDtypeStruct((_NC, _NP, _D), jnp.float32),
                jax.ShapeDtypeStruct((_NC, _NP), jnp.float32)),
      mesh=mesh,
      scratch_types=[
          pltpu.VMEM_SHARED((_NP, _D), jnp.float32),
          pltpu.VMEM_SHARED((_NP,), jnp.float32),
          pltpu.VMEM((_EPT,), jnp.int32),
          pltpu.VMEM((_K,), jnp.int32),
          pltpu.VMEM((_K,), jnp.int32),
          pltpu.VMEM((_K, _D), jnp.float32),
          pltpu.VMEM((_K, _D), jnp.float32),
          pltpu.VMEM((_K,), jnp.float32),
          pltpu.VMEM((_RPT,), jnp.float32),
          pltpu.SemaphoreType.DMA,
          pltpu.SemaphoreType.DMA,
          pltpu.SemaphoreType.DMA,
          pltpu.SemaphoreType.DMA,
      ],
      compiler_params=pltpu.CompilerParams(needs_layout_passes=False),
  )
  def seg_sum(h_hbm, src_hbm, dst_hbm, zrows_hbm,
              agg_out, cnt_out, agg_sp, cnt_sp, idx_all_s,
              idx_d_a, idx_d_b, rows_a, rows_b, ones_v, cbuf,
              sem_ra, sem_rb, sem_ia, sem_ib):
    c = lax.axis_index("c")
    s = lax.axis_index("s")
    r0 = s * _RPT
    base = (c * _NS + s) * _EPT
    pltpu.sync_copy(src_hbm.at[pl.ds(base, _EPT)], idx_all_s)
    zeros16 = jnp.zeros((16,), jnp.float32)
    ones16 = jnp.ones((16,), jnp.float32)
    for j in range(_K // 16):
      ones_v[pl.ds(j * 16, 16)] = ones16
    for j in range(_RPT // 16):
      cbuf[pl.ds(j * 16, 16)] = zeros16
    pltpu.sync_copy(cbuf, cnt_sp.at[pl.ds(r0, _RPT)])
    pltpu.sync_copy(zrows_hbm, rows_a)
    for j in range(_RPT // _K):
      pltpu.sync_copy(rows_a, agg_sp.at[pl.ds(r0 + j * _K, _K)])
    plsc.subcore_barrier()

    def fetch(g, idx_d_buf, rows_buf, sem_r, sem_i):
      pltpu.async_copy(dst_hbm.at[pl.ds(base + g * _K, _K)], idx_d_buf,
                       sem_i)
      pltpu.async_copy(h_hbm.at[idx_all_s.at[pl.ds(g * _K, _K)]],
                       rows_buf, sem_r)

    def wait_chunk(idx_d_buf, rows_buf, sem_r, sem_i):
      pltpu.make_async_copy(dst_hbm.at[pl.ds(0, _K)], idx_d_buf,
                            sem_i).wait()
      pltpu.make_async_copy(h_hbm.at[pl.ds(0, _K)], rows_buf, sem_r).wait()

    fetch(0, idx_d_a, rows_a, sem_ra, sem_ia)
    fetch(1, idx_d_b, rows_b, sem_rb, sem_ib)

    def body(t, carry):
      for bi, (idx_d_buf, rows_buf, sem_r, sem_i) in enumerate(
          ((idx_d_a, rows_a, sem_ra, sem_ia),
           (idx_d_b, rows_b, sem_rb, sem_ib))):
        g = 2 * t + bi
        wait_chunk(idx_d_buf, rows_buf, sem_r, sem_i)
        pltpu.sync_copy(rows_buf, agg_sp.at[idx_d_buf], add=True)
        pltpu.sync_copy(ones_v, cnt_sp.at[idx_d_buf], add=True)
        pg = g + 2

        @pl.when(pg < _NCHUNK)
        def _():
          fetch(pg, idx_d_buf, rows_buf, sem_r, sem_i)

      return carry

    lax.fori_loop(0, _NCHUNK // 2, body, 0)
    wait_chunk(idx_d_a, rows_a, sem_ra, sem_ia)
    pltpu.sync_copy(rows_a, agg_sp.at[idx_d_a], add=True)
    pltpu.sync_copy(ones_v, cnt_sp.at[idx_d_a], add=True)
    plsc.subcore_barrier()

    for j in range(_RPT // _K):
      pltpu.sync_copy(agg_sp.at[pl.ds(r0 + j * _K, _K)], rows_a)
      pltpu.sync_copy(rows_a, agg_out.at[c, pl.ds(r0 + j * _K, _K)])
    pltpu.sync_copy(cnt_sp.at[pl.ds(r0, _RPT)], cbuf)
    pltpu.sync_copy(cbuf, cnt_out.at[c, pl.ds(r0, _RPT)])

  return seg_sum(h, src, dst, zrows)


def _tc_self(h, Wr):
  def body(h_ref, wr_ref, out_ref):
    out_ref[...] = lax.dot_general(h_ref[...], wr_ref[...],
                                   (((1,), (1,)), ((), ())),
                                   preferred_element_type=jnp.float32)

  return pl.pallas_call(
      body,
      out_shape=jax.ShapeDtypeStruct((_N, _D), jnp.float32),
  )(h, Wr)


def _tc_merge(p, cntp, hr, Wl, bl, gw, gb, gms):
  def body(p_ref, cnt_ref, hr_ref, wl_ref, bl_ref, gw_ref, gb_ref,
           gms_ref, out_ref):
    agg = p_ref[0, :_N] + p_ref[1, :_N]
    cnt_row = jnp.sum(cnt_ref[...], axis=0, keepdims=True)
    cnt = jnp.transpose(cnt_row[:, :_N])
    agg = agg / jnp.maximum(cnt, 1.0)
    z = (lax.dot_general(agg, wl_ref[...], (((1,), (1,)), ((), ())),
                         preferred_element_type=jnp.float32)
         + bl_ref[...] + hr_ref[...])
    z = jnp.maximum(z, 0.0)
    mean = jnp.mean(z, axis=0, keepdims=True)
    out = z - mean * gms_ref[...]
    var = jnp.mean(out * out, axis=0, keepdims=True)
    out = out * lax.rsqrt(var + 1e-5)
    out_ref[...] = out * gw_ref[...] + gb_ref[...]

  return pl.pallas_call(
      body,
      out_shape=jax.ShapeDtypeStruct((_N, _D), jnp.float32),
  )(p, cntp, hr, Wl, bl, gw, gb, gms)


def kernel(x, edge_index, W1l, b1l, W1r, W2l, b2l, W2r,
           gn_weight, gn_bias, gn_mean_scale):
  src = edge_index[0].astype(jnp.int32)
  dst = edge_index[1].astype(jnp.int32)
  zrows = jnp.zeros((_K, _D), jnp.float32)
  gw = gn_weight.reshape(1, _D)
  gb = gn_bias.reshape(1, _D)
  gms = gn_mean_scale.reshape(1, _D)
  h = x
  for Wl, bl, Wr in ((W1l, b1l, W1r), (W2l, b2l, W2r)):
    hr = _tc_self(h, Wr)
    p, cntp = _sc_segment_sum(h, src, dst, zrows)
    h = _tc_merge(p, cntp, hr, Wl, bl.reshape(1, _D), gw, gb, gms)
  return h


# speedup vs baseline: 4.0551x; 1.1949x over previous
import functools

import jax
import jax.numpy as jnp
from jax import lax
from jax.experimental import pallas as pl
from jax.experimental.pallas import tpu as pltpu
from jax.experimental.pallas import tpu_sc as plsc

_N = 10000
_E = 320000
_D = 128
_NC = 2
_NS = 16
_NW = _NC * _NS
_EPT = _E // _NW
_K = 80
_NCHUNK = _EPT // _K
_NP = 10240
_RPT = _NP // _NS


def _sc_segment_sum(h, src, dst, zrows):
  mesh = plsc.VectorSubcoreMesh(core_axis_name="c", subcore_axis_name="s",
                                num_cores=_NC, num_subcores=_NS)

  @functools.partial(
      pl.kernel,
      out_type=(jax.ShapeDtypeStruct((_NC, _NP, _D), jnp.float32),
                jax.ShapeDtypeStruct((_NC, _NP), jnp.float32)),
      mesh=mesh,
      scratch_types=[
          pltpu.VMEM_SHARED((_NP, _D), jnp.float32),
          pltpu.VMEM_SHARED((_NP,), jnp.float32),
          pltpu.VMEM((_EPT,), jnp.int32),
          pltpu.VMEM((_K,), jnp.int32),
          pltpu.VMEM((_K,), jnp.int32),
          pltpu.VMEM((_K,), jnp.int32),
          pltpu.VMEM((_K, _D), jnp.float32),
          pltpu.VMEM((_K, _D), jnp.float32),
          pltpu.VMEM((_K, _D), jnp.float32),
          pltpu.VMEM((_K,), jnp.float32),
          pltpu.VMEM((_RPT,), jnp.float32),
          pltpu.SemaphoreType.DMA,
          pltpu.SemaphoreType.DMA,
          pltpu.SemaphoreType.DMA,
          pltpu.SemaphoreType.DMA,
          pltpu.SemaphoreType.DMA,
          pltpu.SemaphoreType.DMA,
      ],
      compiler_params=pltpu.CompilerParams(needs_layout_passes=False),
  )
  def seg_sum(h_hbm, src_hbm, dst_hbm, zrows_hbm,
              agg_out, cnt_out, agg_sp, cnt_sp, idx_all_s,
              idx_d_a, idx_d_b, idx_d_c, rows_a, rows_b, rows_c, ones_v,
              cbuf, sem_ra, sem_rb, sem_rc, sem_ia, sem_ib, sem_ic):
    c = lax.axis_index("c")
    s = lax.axis_index("s")
    r0 = s * _RPT
    base = (c * _NS + s) * _EPT
    pltpu.sync_copy(src_hbm.at[pl.ds(base, _EPT)], idx_all_s)
    zeros16 = jnp.zeros((16,), jnp.float32)
    ones16 = jnp.ones((16,), jnp.float32)
    for j in range(_K // 16):
      ones_v[pl.ds(j * 16, 16)] = ones16
    for j in range(_RPT // 16):
      cbuf[pl.ds(j * 16, 16)] = zeros16
    pltpu.sync_copy(cbuf, cnt_sp.at[pl.ds(r0, _RPT)])
    pltpu.sync_copy(zrows_hbm, rows_a)
    for j in range(_RPT // _K):
      pltpu.sync_copy(rows_a, agg_sp.at[pl.ds(r0 + j * _K, _K)])
    plsc.subcore_barrier()

    def fetch(g, idx_d_buf, rows_buf, sem_r, sem_i):
      pltpu.async_copy(dst_hbm.at[pl.ds(base + g * _K, _K)], idx_d_buf,
                       sem_i)
      pltpu.async_copy(h_hbm.at[idx_all_s.at[pl.ds(g * _K, _K)]],
                       rows_buf, sem_r)

    def wait_chunk(idx_d_buf, rows_buf, sem_r, sem_i):
      pltpu.make_async_copy(dst_hbm.at[pl.ds(0, _K)], idx_d_buf,
                            sem_i).wait()
      pltpu.make_async_copy(h_hbm.at[pl.ds(0, _K)], rows_buf, sem_r).wait()

    bufs = ((idx_d_a, rows_a, sem_ra, sem_ia),
            (idx_d_b, rows_b, sem_rb, sem_ib),
            (idx_d_c, rows_c, sem_rc, sem_ic))
    _NB = len(bufs)
    for g0 in range(_NB):
      fetch(g0, *bufs[g0])

    def process(g, idx_d_buf, rows_buf, sem_r, sem_i):
      wait_chunk(idx_d_buf, rows_buf, sem_r, sem_i)
      pltpu.sync_copy(rows_buf, agg_sp.at[idx_d_buf], add=True)
      pltpu.sync_copy(ones_v, cnt_sp.at[idx_d_buf], add=True)

    def body(t, carry):
      for bi in range(_NB):
        g = _NB * t + bi
        process(g, *bufs[bi])
        pg = g + _NB

        @pl.when(pg < _NCHUNK)
        def _():
          fetch(pg, *bufs[bi])

      return carry

    lax.fori_loop(0, _NCHUNK // _NB, body, 0)
    for g in range(_NCHUNK - _NCHUNK % _NB, _NCHUNK):
      process(g, *bufs[g % _NB])
    plsc.subcore_barrier()

    for j in range(_RPT // _K):
      pltpu.sync_copy(agg_sp.at[pl.ds(r0 + j * _K, _K)], rows_a)
      pltpu.sync_copy(rows_a, agg_out.at[c, pl.ds(r0 + j * _K, _K)])
    pltpu.sync_copy(cnt_sp.at[pl.ds(r0, _RPT)], cbuf)
    pltpu.sync_copy(cbuf, cnt_out.at[c, pl.ds(r0, _RPT)])

  return seg_sum(h, src, dst, zrows)


def _tc_self(h, Wr):
  def body(h_ref, wr_ref, out_ref):
    out_ref[...] = lax.dot_general(h_ref[...], wr_ref[...],
                                   (((1,), (1,)), ((), ())),
                                   preferred_element_type=jnp.float32)

  return pl.pallas_call(
      body,
      out_shape=jax.ShapeDtypeStruct((_N, _D), jnp.float32),
  )(h, Wr)


def _tc_merge(p, cntp, hr, Wl, bl, gw, gb, gms):
  def body(p_ref, cnt_ref, hr_ref, wl_ref, bl_ref, gw_ref, gb_ref,
           gms_ref, out_ref):
    agg = p_ref[0, :_N] + p_ref[1, :_N]
    cnt_row = jnp.sum(cnt_ref[...], axis=0, keepdims=True)
    cnt = jnp.transpose(cnt_row[:, :_N])
    agg = agg / jnp.maximum(cnt, 1.0)
    z = (lax.dot_general(agg, wl_ref[...], (((1,), (1,)), ((), ())),
                         preferred_element_type=jnp.float32)
         + bl_ref[...] + hr_ref[...])
    z = jnp.maximum(z, 0.0)
    mean = jnp.mean(z, axis=0, keepdims=True)
    out = z - mean * gms_ref[...]
    var = jnp.mean(out * out, axis=0, keepdims=True)
    out = out * lax.rsqrt(var + 1e-5)
    out_ref[...] = out * gw_ref[...] + gb_ref[...]

  return pl.pallas_call(
      body,
      out_shape=jax.ShapeDtypeStruct((_N, _D), jnp.float32),
  )(p, cntp, hr, Wl, bl, gw, gb, gms)


def kernel(x, edge_index, W1l, b1l, W1r, W2l, b2l, W2r,
           gn_weight, gn_bias, gn_mean_scale):
  src = edge_index[0].astype(jnp.int32)
  dst = edge_index[1].astype(jnp.int32)
  zrows = jnp.zeros((_K, _D), jnp.float32)
  gw = gn_weight.reshape(1, _D)
  gb = gn_bias.reshape(1, _D)
  gms = gn_mean_scale.reshape(1, _D)
  h = x
  for Wl, bl, Wr in ((W1l, b1l, W1r), (W2l, b2l, W2r)):
    hr = _tc_self(h, Wr)
    p, cntp = _sc_segment_sum(h, src, dst, zrows)
    h = _tc_merge(p, cntp, hr, Wl, bl.reshape(1, _D), gw, gb, gms)
  return h
